# baseline (speedup 1.0000x reference)
import functools

import jax
import jax.numpy as jnp
from jax import lax
from jax.experimental import pallas as pl
from jax.experimental.pallas import tpu as pltpu
from jax.experimental.pallas import tpu_sc as plsc

B = 16384
D = 16
NROW = 1000000
NPAD = 1000064
FBLK = NPAD // 128
NBLK = D * FBLK
NC = 2
NS = 16
NW = NC * NS
BPW = B // NW
CW = 16
NCH = BPW // CW


PCB = 8192
PGRID = -(-NPAD // PCB)


def _pad_body(t_ref, out_ref):
    out_ref[...] = t_ref[...]


def _tc_padcopy(table_t):
    out = pl.pallas_call(
        _pad_body,
        grid=(PGRID,),
        in_specs=[pl.BlockSpec((D, PCB), lambda i: (0, i))],
        out_specs=pl.BlockSpec((D, PCB), lambda i: (0, i)),
        out_shape=jax.ShapeDtypeStruct((D, NPAD), jnp.float32),
    )(table_t)
    return out.reshape(NBLK, 128)


def _sc_gather(user_indices, item_indices, t_mfu, t_mfi, t_mlu, t_mli):
    mesh = plsc.VectorSubcoreMesh(core_axis_name="c", subcore_axis_name="s")

    @functools.partial(
        pl.kernel,
        out_type=[jax.ShapeDtypeStruct((D, B), jnp.float32) for _ in range(4)],
        mesh=mesh,
        scratch_types=(
            [pltpu.VMEM((BPW,), jnp.int32) for _ in range(2)]
            + [pltpu.VMEM((D, CW, 128), jnp.float32) for _ in range(2)]
            + [pltpu.VMEM((D, CW), jnp.int32) for _ in range(2)]
            + [pltpu.VMEM((D, BPW), jnp.float32) for _ in range(4)]
            + [pltpu.SemaphoreType.DMA, pltpu.SemaphoreType.DMA]
        ),
        compiler_params=pltpu.CompilerParams(use_tc_tiling_on_sc=True,
                                             needs_layout_passes=False),
    )
    def gather_k(uidx, iidx, tmfu, tmfi, tmlu, tmli,
                 o_mfu, o_mfi, o_mlu, o_mli,
                 uix, iix, g0, g1, bx0, bx1,
                 r_mfu, r_mfi, r_mlu, r_mli, sem0, sem1):
        wid = lax.axis_index("s") * NC + lax.axis_index("c")
        base = wid * BPW
        pltpu.sync_copy(uidx.at[pl.ds(base, BPW)], uix)
        pltpu.sync_copy(iidx.at[pl.ds(base, BPW)], iix)

        gbufs = (g0, g1)
        bxbufs = (bx0, bx1)
        sems = (sem0, sem1)
        iota16 = lax.iota(jnp.int32, 16)

        def fire(tbl, ix, c, par):
            rvec = ix[pl.ds(c * CW, CW)]
            blk = jnp.right_shift(rvec, 7)
            bx = bxbufs[par]
            for d in range(D):
                bx[d, :] = blk + (d * FBLK)
            hs = []
            for d in range(D):
                hs.append(pltpu.async_copy(
                    tbl.at[bx.at[d]], gbufs[par].at[d], sems[par]))
            return hs

        def extract(ix, rbuf, c, par):
            rvec = ix[pl.ds(c * CW, CW)]
            lanes = jnp.bitwise_and(rvec, 127)
            g = gbufs[par]
            for d in range(D):
                dvec = jnp.full((16,), d, jnp.int32)
                vals = plsc.load_gather(g, [dvec, iota16, lanes])
                rbuf[d, pl.ds(c * CW, CW)] = vals

        for tbl, ix, rbuf in ((tmfu, uix, r_mfu), (tmfi, iix, r_mfi),
                              (tmlu, uix, r_mlu), (tmli, iix, r_mli)):
            def pair(p, carry, _tbl=tbl, _ix=ix, _rb=rbuf):
                c0 = p * 2
                h0 = fire(_tbl, _ix, c0, 0)
                h1 = fire(_tbl, _ix, c0 + 1, 1)
                for h in h0:
                    h.wait()
                extract(_ix, _rb, c0, 0)
                for h in h1:
                    h.wait()
                extract(_ix, _rb, c0 + 1, 1)
                return carry

            lax.fori_loop(0, NCH // 2, pair, 0)

        out_sl = pl.ds(base, BPW)
        for rbuf, o in ((r_mfu, o_mfu), (r_mfi, o_mfi),
                        (r_mlu, o_mlu), (r_mli, o_mli)):
            pltpu.sync_copy(rbuf, o.at[:, out_sl])

    return gather_k(user_indices, item_indices, t_mfu, t_mfi, t_mlu, t_mli)


BB = 2048


def _dense_body(mfu_ref, mfi_ref, mlu_ref, mli_ref, w0_ref, b0_ref, wp_ref,
                bp_ref, out_ref):
    mf = mfu_ref[...] * mfi_ref[...]
    w0 = w0_ref[...]
    h = (jnp.dot(w0[:, :D], mlu_ref[...],
                 preferred_element_type=jnp.float32)
         + jnp.dot(w0[:, D:], mli_ref[...],
                   preferred_element_type=jnp.float32))
    h = jnp.maximum(h + b0_ref[...], 0.0)
    wp = wp_ref[...]
    logit = (jnp.sum(mf * wp[:, :D].reshape(D, 1), axis=0)
             + jnp.sum(h * wp[:, D:].reshape(D, 1), axis=0)
             + bp_ref[0, 0])
    out_ref[...] = jax.nn.sigmoid(logit).reshape(1, 1, BB)


def _tc_dense(mfu, mfi, mlu, mli, W0, b0, Wp, bp):
    nblk = B // BB
    col_spec = pl.BlockSpec((D, BB), lambda i: (0, i))
    full = lambda shape: pl.BlockSpec(shape, lambda i: (0,) * len(shape))
    out2d = pl.pallas_call(
        _dense_body,
        grid=(nblk,),
        in_specs=[col_spec, col_spec, col_spec, col_spec,
                  full((D, 2 * D)), full((D, 1)), full((1, 2 * D)),
                  full((1, 1))],
        out_specs=pl.BlockSpec((1, 1, BB), lambda i: (i, 0, 0)),
        out_shape=jax.ShapeDtypeStruct((nblk, 1, BB), jnp.float32),
    )(mfu, mfi, mlu, mli, W0, b0.reshape(D, 1), Wp, bp.reshape(1, 1))
    return out2d.reshape(B)


def kernel(user_indices, item_indices, mf_emb_user, mf_emb_item,
           mlp_emb_user, mlp_emb_item, W0, b0, Wp, bp):
    views = [_tc_padcopy(t.T)
             for t in (mf_emb_user, mf_emb_item, mlp_emb_user, mlp_emb_item)]
    mfu, mfi, mlu, mli = _sc_gather(
        user_indices.astype(jnp.int32), item_indices.astype(jnp.int32),
        *views)
    return _tc_dense(mfu, mfi, mlu, mli, W0, b0, Wp, bp)


# speedup vs baseline: 1.3782x; 1.3782x over previous
import functools

import jax
import jax.numpy as jnp
from jax import lax
from jax.experimental import pallas as pl
from jax.experimental.pallas import tpu as pltpu
from jax.experimental.pallas import tpu_sc as plsc

B = 16384
D = 16
NROW = 1000000
NPAD = 1000064
FBLK = NPAD // 128
NBLK = D * FBLK
NC = 2
NS = 16
NW = NC * NS
BPW = B // NW
CW = 16
NCH = BPW // CW


PCB = 8192
PGRID = -(-NPAD // PCB)


def _pad_body(t_ref, out_ref):
    out_ref[...] = t_ref[...]


def _tc_padcopy(table_t):
    out = pl.pallas_call(
        _pad_body,
        grid=(PGRID,),
        in_specs=[pl.BlockSpec((D, PCB), lambda i: (0, i))],
        out_specs=pl.BlockSpec((D, PCB), lambda i: (0, i)),
        out_shape=jax.ShapeDtypeStruct((D, NPAD), jnp.float32),
    )(table_t)
    return out.reshape(NBLK, 128)


def _sc_gather1(indices, table_view):
    mesh = plsc.VectorSubcoreMesh(core_axis_name="c", subcore_axis_name="s")

    @functools.partial(
        pl.kernel,
        out_type=jax.ShapeDtypeStruct((D, B), jnp.float32),
        mesh=mesh,
        scratch_types=(
            [pltpu.VMEM((BPW,), jnp.int32)]
            + [pltpu.VMEM((D, CW, 128), jnp.float32) for _ in range(2)]
            + [pltpu.VMEM((D, CW), jnp.int32) for _ in range(2)]
            + [pltpu.VMEM((D, BPW), jnp.float32)]
            + [pltpu.SemaphoreType.DMA, pltpu.SemaphoreType.DMA]
        ),
        compiler_params=pltpu.CompilerParams(use_tc_tiling_on_sc=True,
                                             needs_layout_passes=False),
    )
    def gather_k(idx, tbl, o, ix, g0, g1, bx0, bx1, rbuf, sem0, sem1):
        wid = lax.axis_index("s") * NC + lax.axis_index("c")
        base = wid * BPW
        pltpu.sync_copy(idx.at[pl.ds(base, BPW)], ix)

        gbufs = (g0, g1)
        bxbufs = (bx0, bx1)
        sems = (sem0, sem1)
        iota16 = lax.iota(jnp.int32, 16)

        def fire(c, par):
            rvec = ix[pl.ds(c * CW, CW)]
            blk = jnp.right_shift(rvec, 7)
            bx = bxbufs[par]
            for d in range(D):
                bx[d, :] = blk + (d * FBLK)
            return [pltpu.async_copy(tbl.at[bx.at[d]], gbufs[par].at[d],
                                     sems[par]) for d in range(D)]

        def extract(c, par):
            rvec = ix[pl.ds(c * CW, CW)]
            lanes = jnp.bitwise_and(rvec, 127)
            g = gbufs[par]
            for d in range(D):
                dvec = jnp.full((16,), d, jnp.int32)
                rbuf[d, pl.ds(c * CW, CW)] = plsc.load_gather(
                    g, [dvec, iota16, lanes])

        def pair(p, carry):
            c0 = p * 2
            h0 = fire(c0, 0)
            h1 = fire(c0 + 1, 1)
            for h in h0:
                h.wait()
            extract(c0, 0)
            for h in h1:
                h.wait()
            extract(c0 + 1, 1)
            return carry

        lax.fori_loop(0, NCH // 2, pair, 0)
        pltpu.sync_copy(rbuf, o.at[:, pl.ds(base, BPW)])

    return gather_k(indices, table_view)


BB = 2048


def _dense_body(mfu_ref, mfi_ref, mlu_ref, mli_ref, w0_ref, b0_ref, wp_ref,
                bp_ref, out_ref):
    mf = mfu_ref[...] * mfi_ref[...]
    w0 = w0_ref[...]
    h = (jnp.dot(w0[:, :D], mlu_ref[...],
                 preferred_element_type=jnp.float32)
         + jnp.dot(w0[:, D:], mli_ref[...],
                   preferred_element_type=jnp.float32))
    h = jnp.maximum(h + b0_ref[...], 0.0)
    wp = wp_ref[...]
    logit = (jnp.sum(mf * wp[:, :D].reshape(D, 1), axis=0)
             + jnp.sum(h * wp[:, D:].reshape(D, 1), axis=0)
             + bp_ref[0, 0])
    out_ref[...] = jax.nn.sigmoid(logit).reshape(1, 1, BB)


def _tc_dense(mfu, mfi, mlu, mli, W0, b0, Wp, bp):
    nblk = B // BB
    col_spec = pl.BlockSpec((D, BB), lambda i: (0, i))
    full = lambda shape: pl.BlockSpec(shape, lambda i: (0,) * len(shape))
    out2d = pl.pallas_call(
        _dense_body,
        grid=(nblk,),
        in_specs=[col_spec, col_spec, col_spec, col_spec,
                  full((D, 2 * D)), full((D, 1)), full((1, 2 * D)),
                  full((1, 1))],
        out_specs=pl.BlockSpec((1, 1, BB), lambda i: (i, 0, 0)),
        out_shape=jax.ShapeDtypeStruct((nblk, 1, BB), jnp.float32),
    )(mfu, mfi, mlu, mli, W0, b0.reshape(D, 1), Wp, bp.reshape(1, 1))
    return out2d.reshape(B)


def kernel(user_indices, item_indices, mf_emb_user, mf_emb_item,
           mlp_emb_user, mlp_emb_item, W0, b0, Wp, bp):
    ui = user_indices.astype(jnp.int32)
    ii = item_indices.astype(jnp.int32)
    views = [jnp.pad(t.T, ((0, 0), (0, NPAD - NROW))).reshape(NBLK, 128)
             for t in (mf_emb_user, mf_emb_item, mlp_emb_user, mlp_emb_item)]
    mfu = _sc_gather1(ui, views[0])
    mfi = _sc_gather1(ii, views[1])
    mlu = _sc_gather1(ui, views[2])
    mli = _sc_gather1(ii, views[3])
    return _tc_dense(mfu, mfi, mlu, mli, W0, b0, Wp, bp)
